# baseline (speedup 1.0000x reference)
import functools

import jax
import jax.numpy as jnp
from jax.experimental import pallas as pl
from jax.experimental.pallas import tpu as pltpu

_E_BLK = 2000


def _dense_body(x_ref, w_ref, ms_ref, md_ref, ht_ref, ad_ref):
    h = jnp.dot(x_ref[...], w_ref[...], preferred_element_type=jnp.float32)
    ht_ref[:, : h.shape[1]] = h
    ht_ref[:, h.shape[1] :] = jnp.dot(h, ms_ref[...], preferred_element_type=jnp.float32)
    ad_ref[...] = jnp.dot(h, md_ref[...], preferred_element_type=jnp.float32)


def _dense(x, w, ms, md):
    n, din = x.shape
    dout = w.shape[1]
    blk = 1000
    return pl.pallas_call(
        _dense_body,
        grid=(n // blk,),
        in_specs=[
            pl.BlockSpec((blk, din), lambda i: (i, 0)),
            pl.BlockSpec((din, dout), lambda i: (0, 0)),
            pl.BlockSpec((dout, dout), lambda i: (0, 0)),
            pl.BlockSpec((dout, dout), lambda i: (0, 0)),
        ],
        out_specs=[
            pl.BlockSpec((blk, 2 * dout), lambda i: (i, 0)),
            pl.BlockSpec((blk, dout), lambda i: (i, 0)),
        ],
        out_shape=[
            jax.ShapeDtypeStruct((n, 2 * dout), jnp.float32),
            jax.ShapeDtypeStruct((n, dout), jnp.float32),
        ],
    )(x, w, ms, md)


def _edge_body(src_ref, dst_ref, ht_ref, ad_ref, ud_ref, *, blk, dout):
    step = pl.program_id(0)

    @pl.when(step == 0)
    def _init():
        ud_ref[...] = jnp.zeros_like(ud_ref)

    def body(j, carry):
        s = src_ref[0, 0, j]
        d = dst_ref[0, 0, j]
        row = ht_ref[pl.ds(s, 1), :]
        e = row[:, dout:] + ad_ref[pl.ds(d, 1), :]
        e = jnp.where(e >= 0.0, e, 0.2 * e)
        w = jnp.exp(e)
        upd = jnp.concatenate([w * row[:, :dout], w], axis=1)
        ud_ref[pl.ds(d, 1), :] += upd
        return carry

    jax.lax.fori_loop(0, blk, body, 0)


def _edge(ht, adx, src3, dst3):
    n, dout2 = ht.shape
    dout = dout2 // 2
    nb, _, blk = src3.shape
    return pl.pallas_call(
        functools.partial(_edge_body, blk=blk, dout=dout),
        grid=(nb,),
        in_specs=[
            pl.BlockSpec((1, 1, blk), lambda i: (i, 0, 0), memory_space=pltpu.SMEM),
            pl.BlockSpec((1, 1, blk), lambda i: (i, 0, 0), memory_space=pltpu.SMEM),
            pl.BlockSpec((n, dout2), lambda i: (0, 0)),
            pl.BlockSpec((n, dout), lambda i: (0, 0)),
        ],
        out_specs=pl.BlockSpec((n, dout2), lambda i: (0, 0)),
        out_shape=jax.ShapeDtypeStruct((n, dout2), jnp.float32),
    )(src3, dst3, ht, adx)


def _norm_body(ud_ref, b_ref, o_ref, *, relu, dout):
    y = ud_ref[:, :dout] / (ud_ref[:, dout:] + 1e-16) + b_ref[...]
    if relu:
        y = jnp.maximum(y, 0.0)
    o_ref[...] = y


def _norm(ud, b, relu):
    n, dout2 = ud.shape
    dout = dout2 // 2
    blk = 1000
    return pl.pallas_call(
        functools.partial(_norm_body, relu=relu, dout=dout),
        grid=(n // blk,),
        in_specs=[
            pl.BlockSpec((blk, dout2), lambda i: (i, 0)),
            pl.BlockSpec((1, dout), lambda i: (0, 0)),
        ],
        out_specs=pl.BlockSpec((blk, dout), lambda i: (i, 0)),
        out_shape=jax.ShapeDtypeStruct((n, dout), jnp.float32),
    )(ud, b)


def _fold_attn(a, heads, c):
    a2 = a.reshape(heads, c)
    eye = jnp.eye(heads, dtype=a.dtype)
    base = (a2[:, :, None] * eye[:, None, :]).reshape(heads * c, heads)
    head_of_col = jnp.arange(heads * c, dtype=jnp.int32) // c
    return jnp.take(base, head_of_col, axis=1)


def _gat_layer(x, src3, dst3, w, a_src, a_dst, b, heads, c, relu):
    ms = _fold_attn(a_src, heads, c)
    md = _fold_attn(a_dst, heads, c)
    ht, adx = _dense(x, w, ms, md)
    ud = _edge(ht, adx, src3, dst3)
    return _norm(ud, b.reshape(1, heads * c), relu)


def kernel(x, edge_index, W1, a_src1, a_dst1, b1, W2, a_src2, a_dst2, b2):
    n = x.shape[0]
    loop = jnp.arange(n, dtype=jnp.int32)
    src = jnp.concatenate([edge_index[0].astype(jnp.int32), loop])
    dst = jnp.concatenate([edge_index[1].astype(jnp.int32), loop])
    ne = src.shape[0]
    nb = ne // _E_BLK
    src3 = src.reshape(nb, 1, _E_BLK)
    dst3 = dst.reshape(nb, 1, _E_BLK)

    heads1, c1 = a_src1.shape[1], a_src1.shape[2]
    y = _gat_layer(x, src3, dst3, W1, a_src1, a_dst1, b1, heads1, c1, relu=True)
    heads2, c2 = a_src2.shape[1], a_src2.shape[2]
    out = _gat_layer(y, src3, dst3, W2, a_src2, a_dst2, b2, heads2, c2, relu=False)
    return out


# speedup vs baseline: 2.7656x; 2.7656x over previous
import functools

import jax
import jax.numpy as jnp
from jax.experimental import pallas as pl
from jax.experimental.pallas import tpu as pltpu

_E_BLK = 2000


def _dense_body(x_ref, w_ref, ms_ref, md_ref, h_ref, as_ref, ad_ref):
    h = jnp.dot(x_ref[...], w_ref[...], preferred_element_type=jnp.float32)
    h_ref[...] = h
    as_ref[...] = jnp.dot(h, ms_ref[...], preferred_element_type=jnp.float32)
    ad_ref[...] = jnp.dot(h, md_ref[...], preferred_element_type=jnp.float32)


def _dense(x, w, ms, md):
    n, din = x.shape
    dout = w.shape[1]
    blk = 1000
    return pl.pallas_call(
        _dense_body,
        grid=(n // blk,),
        in_specs=[
            pl.BlockSpec((blk, din), lambda i: (i, 0)),
            pl.BlockSpec((din, dout), lambda i: (0, 0)),
            pl.BlockSpec((dout, dout), lambda i: (0, 0)),
            pl.BlockSpec((dout, dout), lambda i: (0, 0)),
        ],
        out_specs=[
            pl.BlockSpec((blk, dout), lambda i: (i, 0)),
            pl.BlockSpec((blk, dout), lambda i: (i, 0)),
            pl.BlockSpec((blk, dout), lambda i: (i, 0)),
        ],
        out_shape=[
            jax.ShapeDtypeStruct((n, dout), jnp.float32),
            jax.ShapeDtypeStruct((n, dout), jnp.float32),
            jax.ShapeDtypeStruct((n, dout), jnp.float32),
        ],
    )(x, w, ms, md)


def _edge_body(src_ref, dst_ref, h_ref, as_ref, ad_ref, u_ref, den_ref, *, blk):
    step = pl.program_id(0)

    @pl.when(step == 0)
    def _init():
        u_ref[...] = jnp.zeros_like(u_ref)
        den_ref[...] = jnp.zeros_like(den_ref)

    def body(j, carry):
        s = src_ref[0, 0, j]
        d = dst_ref[0, 0, j]
        e = as_ref[pl.ds(s, 1), :] + ad_ref[pl.ds(d, 1), :]
        e = jnp.where(e >= 0.0, e, 0.2 * e)
        w = jnp.exp(e)
        u_ref[pl.ds(d, 1), :] += w * h_ref[pl.ds(s, 1), :]
        den_ref[pl.ds(d, 1), :] += w
        return carry

    jax.lax.fori_loop(0, blk, body, 0, unroll=4)


def _edge(h, asx, adx, src3, dst3):
    n, dout = h.shape
    nb, _, blk = src3.shape
    return pl.pallas_call(
        functools.partial(_edge_body, blk=blk),
        grid=(nb,),
        in_specs=[
            pl.BlockSpec((1, 1, blk), lambda i: (i, 0, 0), memory_space=pltpu.SMEM),
            pl.BlockSpec((1, 1, blk), lambda i: (i, 0, 0), memory_space=pltpu.SMEM),
            pl.BlockSpec((n, dout), lambda i: (0, 0)),
            pl.BlockSpec((n, dout), lambda i: (0, 0)),
            pl.BlockSpec((n, dout), lambda i: (0, 0)),
        ],
        out_specs=[
            pl.BlockSpec((n, dout), lambda i: (0, 0)),
            pl.BlockSpec((n, dout), lambda i: (0, 0)),
        ],
        out_shape=[
            jax.ShapeDtypeStruct((n, dout), jnp.float32),
            jax.ShapeDtypeStruct((n, dout), jnp.float32),
        ],
    )(src3, dst3, h, asx, adx)


def _norm_body(u_ref, den_ref, b_ref, o_ref, *, relu):
    y = u_ref[...] / (den_ref[...] + 1e-16) + b_ref[...]
    if relu:
        y = jnp.maximum(y, 0.0)
    o_ref[...] = y


def _norm(u, den, b, relu):
    n, dout = u.shape
    blk = 1000
    return pl.pallas_call(
        functools.partial(_norm_body, relu=relu),
        grid=(n // blk,),
        in_specs=[
            pl.BlockSpec((blk, dout), lambda i: (i, 0)),
            pl.BlockSpec((blk, dout), lambda i: (i, 0)),
            pl.BlockSpec((1, dout), lambda i: (0, 0)),
        ],
        out_specs=pl.BlockSpec((blk, dout), lambda i: (i, 0)),
        out_shape=jax.ShapeDtypeStruct((n, dout), jnp.float32),
    )(u, den, b)


def _fold_attn(a, heads, c):
    a2 = a.reshape(heads, c)
    eye = jnp.eye(heads, dtype=a.dtype)
    base = (a2[:, :, None] * eye[:, None, :]).reshape(heads * c, heads)
    head_of_col = jnp.arange(heads * c, dtype=jnp.int32) // c
    return jnp.take(base, head_of_col, axis=1)


def _gat_layer(x, src3, dst3, w, a_src, a_dst, b, heads, c, relu):
    ms = _fold_attn(a_src, heads, c)
    md = _fold_attn(a_dst, heads, c)
    h, asx, adx = _dense(x, w, ms, md)
    u, den = _edge(h, asx, adx, src3, dst3)
    return _norm(u, den, b.reshape(1, heads * c), relu)


def kernel(x, edge_index, W1, a_src1, a_dst1, b1, W2, a_src2, a_dst2, b2):
    n = x.shape[0]
    loop = jnp.arange(n, dtype=jnp.int32)
    src = jnp.concatenate([edge_index[0].astype(jnp.int32), loop])
    dst = jnp.concatenate([edge_index[1].astype(jnp.int32), loop])
    ne = src.shape[0]
    nb = ne // _E_BLK
    src3 = src.reshape(nb, 1, _E_BLK)
    dst3 = dst.reshape(nb, 1, _E_BLK)

    heads1, c1 = a_src1.shape[1], a_src1.shape[2]
    y = _gat_layer(x, src3, dst3, W1, a_src1, a_dst1, b1, heads1, c1, relu=True)
    heads2, c2 = a_src2.shape[1], a_src2.shape[2]
    out = _gat_layer(y, src3, dst3, W2, a_src2, a_dst2, b2, heads2, c2, relu=False)
    return out


# speedup vs baseline: 3.2375x; 1.1706x over previous
import functools

import jax
import jax.numpy as jnp
from jax.experimental import pallas as pl
from jax.experimental.pallas import tpu as pltpu

_E_BLK = 2000


def _dense_body(x_ref, w_ref, ms_ref, md_ref, h_ref, as_ref, ad_ref):
    h = jnp.dot(x_ref[...], w_ref[...], preferred_element_type=jnp.float32)
    h_ref[...] = h
    as_ref[...] = jnp.dot(h, ms_ref[...], preferred_element_type=jnp.float32)
    ad_ref[...] = jnp.dot(h, md_ref[...], preferred_element_type=jnp.float32)


def _dense(x, w, ms, md):
    n, din = x.shape
    dout = w.shape[1]
    blk = 1000
    return pl.pallas_call(
        _dense_body,
        grid=(n // blk,),
        in_specs=[
            pl.BlockSpec((blk, din), lambda i: (i, 0)),
            pl.BlockSpec((din, dout), lambda i: (0, 0)),
            pl.BlockSpec((dout, dout), lambda i: (0, 0)),
            pl.BlockSpec((dout, dout), lambda i: (0, 0)),
        ],
        out_specs=[
            pl.BlockSpec((blk, dout), lambda i: (i, 0)),
            pl.BlockSpec((blk, dout), lambda i: (i, 0)),
            pl.BlockSpec((blk, dout), lambda i: (i, 0)),
        ],
        out_shape=[
            jax.ShapeDtypeStruct((n, dout), jnp.float32),
            jax.ShapeDtypeStruct((n, dout), jnp.float32),
            jax.ShapeDtypeStruct((n, dout), jnp.float32),
        ],
    )(x, w, ms, md)


def _edge_body(src_ref, dst_ref, h_ref, as_ref, ad_ref, u_ref, den_ref, *, blk):
    step = pl.program_id(0)

    @pl.when(step == 0)
    def _init():
        u_ref[...] = jnp.zeros_like(u_ref)
        den_ref[...] = jnp.zeros_like(den_ref)

    def body(j, carry):
        s = src_ref[0, 0, j]
        d = dst_ref[0, 0, j]
        e = as_ref[pl.ds(s, 1), :] + ad_ref[pl.ds(d, 1), :]
        e = jnp.where(e >= 0.0, e, 0.2 * e)
        w = jnp.exp(e)
        u_ref[pl.ds(d, 1), :] += w * h_ref[pl.ds(s, 1), :]
        den_ref[pl.ds(d, 1), :] += w
        return carry

    jax.lax.fori_loop(0, blk, body, 0, unroll=8)


def _edge(h, asx, adx, src3, dst3):
    n, dout = h.shape
    nb, _, blk = src3.shape
    return pl.pallas_call(
        functools.partial(_edge_body, blk=blk),
        grid=(nb,),
        in_specs=[
            pl.BlockSpec((1, 1, blk), lambda i: (i, 0, 0), memory_space=pltpu.SMEM),
            pl.BlockSpec((1, 1, blk), lambda i: (i, 0, 0), memory_space=pltpu.SMEM),
            pl.BlockSpec((n, dout), lambda i: (0, 0)),
            pl.BlockSpec((n, dout), lambda i: (0, 0)),
            pl.BlockSpec((n, dout), lambda i: (0, 0)),
        ],
        out_specs=[
            pl.BlockSpec((n, dout), lambda i: (0, 0)),
            pl.BlockSpec((n, dout), lambda i: (0, 0)),
        ],
        out_shape=[
            jax.ShapeDtypeStruct((n, dout), jnp.float32),
            jax.ShapeDtypeStruct((n, dout), jnp.float32),
        ],
    )(src3, dst3, h, asx, adx)


def _norm_body(u_ref, den_ref, b_ref, o_ref, *, relu):
    y = u_ref[...] / (den_ref[...] + 1e-16) + b_ref[...]
    if relu:
        y = jnp.maximum(y, 0.0)
    o_ref[...] = y


def _norm(u, den, b, relu):
    n, dout = u.shape
    blk = 1000
    return pl.pallas_call(
        functools.partial(_norm_body, relu=relu),
        grid=(n // blk,),
        in_specs=[
            pl.BlockSpec((blk, dout), lambda i: (i, 0)),
            pl.BlockSpec((blk, dout), lambda i: (i, 0)),
            pl.BlockSpec((1, dout), lambda i: (0, 0)),
        ],
        out_specs=pl.BlockSpec((blk, dout), lambda i: (i, 0)),
        out_shape=jax.ShapeDtypeStruct((n, dout), jnp.float32),
    )(u, den, b)


def _fold_attn(a, heads, c):
    a2 = a.reshape(heads, c)
    eye = jnp.eye(heads, dtype=a.dtype)
    base = (a2[:, :, None] * eye[:, None, :]).reshape(heads * c, heads)
    head_of_col = jnp.arange(heads * c, dtype=jnp.int32) // c
    return jnp.take(base, head_of_col, axis=1)


def _gat_layer(x, src3, dst3, w, a_src, a_dst, b, heads, c, relu):
    ms = _fold_attn(a_src, heads, c)
    md = _fold_attn(a_dst, heads, c)
    h, asx, adx = _dense(x, w, ms, md)
    u, den = _edge(h, asx, adx, src3, dst3)
    return _norm(u, den, b.reshape(1, heads * c), relu)


def kernel(x, edge_index, W1, a_src1, a_dst1, b1, W2, a_src2, a_dst2, b2):
    n = x.shape[0]
    loop = jnp.arange(n, dtype=jnp.int32)
    src = jnp.concatenate([edge_index[0].astype(jnp.int32), loop])
    dst = jnp.concatenate([edge_index[1].astype(jnp.int32), loop])
    ne = src.shape[0]
    nb = ne // _E_BLK
    src3 = src.reshape(nb, 1, _E_BLK)
    dst3 = dst.reshape(nb, 1, _E_BLK)

    heads1, c1 = a_src1.shape[1], a_src1.shape[2]
    y = _gat_layer(x, src3, dst3, W1, a_src1, a_dst1, b1, heads1, c1, relu=True)
    heads2, c2 = a_src2.shape[1], a_src2.shape[2]
    out = _gat_layer(y, src3, dst3, W2, a_src2, a_dst2, b2, heads2, c2, relu=False)
    return out
